# Initial kernel scaffold; baseline (speedup 1.0000x reference)
import jax
import jax.numpy as jnp
from jax.experimental import pallas as pl


def kernel(lhs_embedding, rhs_gnn_embedding, lhs_idgnn_batch, rhs_idgnn_index, rhs_table, W_q, W_k, W_v, W_o, pos_emb, ln_scale, ln_bias, W_proj, b_proj, W_off_emb, b_off_emb, W_off_id, b_off_id, W_head, b_head):
    raise NotImplementedError("write your pallas kernel here")


# speedup vs baseline: 1.6462x; 1.6462x over previous
import functools
import math

import jax
import jax.numpy as jnp
from jax import lax
from jax.experimental import pallas as pl
from jax.experimental.pallas import tpu as pltpu
from jax.experimental.pallas import tpu_sc as plsc

SPW = 8
WIN = 256
NWK = 32
CHW = 128


def _attn_window_kernel(N, L, scale, starts_ref, rhs_hbm, lhs_ref, wq_ref,
                        wk_ref, wv_ref, wo_ref, pe_ref, lns_ref, lnb_ref,
                        wproj_ref, bproj_ref, woffid_ref, boffid_ref,
                        whead_ref, bhead_ref, vals_ref, proj_ref, xw_ref,
                        sem):
    w = pl.program_id(0)
    b0 = w * SPW
    s = jnp.minimum((starts_ref[b0] // 8) * 8, N - WIN)
    cp = pltpu.make_async_copy(rhs_hbm.at[pl.ds(s, WIN)], xw_ref, sem)
    cp.start()
    cp.wait()
    xw = xw_ref[...]
    row = lax.broadcasted_iota(jnp.int32, (WIN, 1), 0)
    roww = lax.broadcasted_iota(jnp.int32, (WIN, WIN), 0)
    colw = lax.broadcasted_iota(jnp.int32, (WIN, WIN), 1)
    mask = jnp.zeros((WIN, WIN), jnp.bool_)
    pos = jnp.zeros((WIN, 1), jnp.int32)
    lseg = jnp.zeros((WIN, 1), jnp.int32)
    for g in range(SPW):
        lo = starts_ref[b0 + g] - s
        hi = starts_ref[b0 + g + 1] - s
        rin = (row >= lo) & (row < hi)
        mask = mask | ((roww >= lo) & (roww < hi)
                       & (colw >= lo) & (colw < hi))
        pos = jnp.where(rin, row - lo, pos)
        lseg = jnp.where(rin, g, lseg)
    pos = jnp.minimum(pos, L - 1)
    lane = lax.broadcasted_iota(jnp.int32, (WIN, L), 1)
    oh_pos = (pos == lane).astype(jnp.float32)
    x = xw + jnp.dot(oh_pos, pe_ref[...],
                     preferred_element_type=jnp.float32)
    q = jnp.dot(x, wq_ref[...], preferred_element_type=jnp.float32)
    k = jnp.dot(x, wk_ref[...], preferred_element_type=jnp.float32)
    v = jnp.dot(x, wv_ref[...], preferred_element_type=jnp.float32)
    scores = lax.dot_general(q, k, (((1,), (1,)), ((), ())),
                             preferred_element_type=jnp.float32) * scale
    scores = jnp.where(mask, scores, -1e9)
    m = jnp.max(scores, axis=-1, keepdims=True)
    p = jnp.exp(scores - m)
    attn = p / jnp.sum(p, axis=-1, keepdims=True)
    out = jnp.dot(jnp.dot(attn, v, preferred_element_type=jnp.float32),
                  wo_ref[...], preferred_element_type=jnp.float32)
    h = xw + out
    mu = jnp.mean(h, axis=-1, keepdims=True)
    var = jnp.mean((h - mu) ** 2, axis=-1, keepdims=True)
    y = (h - mu) * lax.rsqrt(var + 1e-5) * lns_ref[...] + lnb_ref[...]
    lhs8 = lhs_ref[...]
    proj8 = jnp.dot(lhs8, wproj_ref[...],
                    preferred_element_type=jnp.float32) + bproj_ref[...]
    proj_ref[...] = proj8
    offid8 = jnp.dot(proj8, woffid_ref[...],
                     preferred_element_type=jnp.float32) + boffid_ref[0, 0]
    gl = lax.broadcasted_iota(jnp.int32, (WIN, SPW), 1)
    oh_seg = (lseg == gl).astype(jnp.float32)
    lhs_g = jnp.dot(oh_seg, lhs8, preferred_element_type=jnp.float32)
    offid_g = jnp.dot(oh_seg, offid8, preferred_element_type=jnp.float32)
    idgnn = (jnp.dot(y, whead_ref[...], preferred_element_type=jnp.float32)
             + bhead_ref[0, 0]
             + jnp.sum(lhs_g * y, axis=-1, keepdims=True)
             + offid_g)
    vals_ref[...] = idgnn


def _logits_kernel(proj_ref, tab_ref, woffemb_ref, boffemb_ref, out_ref):
    proj = proj_ref[...]
    logits = lax.dot_general(proj, tab_ref[...], (((1,), (1,)), ((), ())),
                             preferred_element_type=jnp.float32)
    off = jnp.dot(proj, woffemb_ref[...],
                  preferred_element_type=jnp.float32) + boffemb_ref[0, 0]
    out_ref[...] = logits + off


def _sc_scatter_body(nch, emb_ref, src_hbm, dst_hbm, vals_hbm,
                     src_v, dst_v, val_v, sem):
    c = lax.axis_index("c")
    s = lax.axis_index("s")
    wid = s * 2 + c
    pltpu.sync_copy(src_hbm.at[wid], src_v)
    pltpu.sync_copy(dst_hbm.at[wid], dst_v)
    for j in range(nch):
        pltpu.async_copy(vals_hbm.at[src_v.at[j]], val_v.at[j], sem).wait()
        pltpu.sync_copy(val_v.at[j], emb_ref.at[dst_v.at[j]])


def kernel(lhs_embedding, rhs_gnn_embedding, lhs_idgnn_batch,
           rhs_idgnn_index, rhs_table, W_q, W_k, W_v, W_o, pos_emb,
           ln_scale, ln_bias, W_proj, b_proj, W_off_emb, b_off_emb,
           W_off_id, b_off_id, W_head, b_head):
    B, C = lhs_embedding.shape
    N = rhs_gnn_embedding.shape[0]
    V, D = rhs_table.shape
    L = pos_emb.shape[0]
    NWIN = B // SPW

    seg = lhs_idgnn_batch.astype(jnp.int32)
    idx = rhs_idgnn_index.astype(jnp.int32)
    starts = jnp.searchsorted(
        seg, jnp.arange(B + 1, dtype=jnp.int32), side='left'
    ).astype(jnp.int32)

    full = lambda shp: pl.BlockSpec(shp, lambda w: tuple(0 for _ in shp))
    vals, proj = pl.pallas_call(
        functools.partial(_attn_window_kernel, N, L, 1.0 / math.sqrt(C)),
        grid=(NWIN,),
        in_specs=[
            pl.BlockSpec(memory_space=pltpu.SMEM),
            pl.BlockSpec(memory_space=pltpu.HBM),
            pl.BlockSpec((SPW, C), lambda w: (w, 0)),
            full((C, C)), full((C, C)), full((C, C)), full((C, C)),
            full((L, C)),
            full((1, C)), full((1, C)),
            full((C, D)), full((1, D)),
            full((D, 1)), full((1, 1)),
            full((C, 1)), full((1, 1)),
        ],
        out_specs=[
            pl.BlockSpec((WIN, 1), lambda w: (w, 0)),
            pl.BlockSpec((SPW, D), lambda w: (w, 0)),
        ],
        out_shape=[
            jax.ShapeDtypeStruct((NWIN * WIN, 1), jnp.float32),
            jax.ShapeDtypeStruct((B, D), jnp.float32),
        ],
        scratch_shapes=[pltpu.VMEM((WIN, C), jnp.float32),
                        pltpu.SemaphoreType.DMA],
        compiler_params=pltpu.CompilerParams(
            dimension_semantics=("arbitrary",)),
    )(starts, rhs_gnn_embedding, lhs_embedding, W_q, W_k, W_v, W_o,
      pos_emb, ln_scale.reshape(1, C), ln_bias.reshape(1, C), W_proj,
      b_proj.reshape(1, D), W_off_id.reshape(D, 1),
      b_off_id.reshape(1, 1), W_head.reshape(C, 1), b_head.reshape(1, 1))

    TB = min(256, B)
    TV = 2048
    NI = B // TB
    NJ = -(-V // TV)
    emb = pl.pallas_call(
        _logits_kernel,
        grid=(NJ, NI),
        in_specs=[
            pl.BlockSpec((TB, D), lambda j, i: (i, 0)),
            pl.BlockSpec((TV, D), lambda j, i: (j, 0)),
            pl.BlockSpec((D, 1), lambda j, i: (0, 0)),
            pl.BlockSpec((1, 1), lambda j, i: (0, 0)),
        ],
        out_specs=pl.BlockSpec((TB, TV), lambda j, i: (i, j)),
        out_shape=jax.ShapeDtypeStruct((B, V), jnp.float32),
        compiler_params=pltpu.CompilerParams(
            dimension_semantics=("parallel", "parallel")),
    )(proj, rhs_table, W_off_emb.reshape(D, 1), b_off_emb.reshape(1, 1))

    n = jnp.arange(N, dtype=jnp.int32)
    w_of = seg // SPW
    ws = jnp.minimum((starts[w_of * SPW] // 8) * 8, N - WIN)
    src = w_of * WIN + n - ws
    dst = seg * V + idx
    NE = -(-N // (NWK * CHW)) * (NWK * CHW)
    if NE != N:
        pad = NE - N
        src = jnp.concatenate([src, jnp.broadcast_to(src[0], (pad,))])
        dst = jnp.concatenate([dst, jnp.broadcast_to(dst[0], (pad,))])
    NCH = NE // (NWK * CHW)
    src3 = src.reshape(NWK, NCH, CHW)
    dst3 = dst.reshape(NWK, NCH, CHW)

    scatter = pl.kernel(
        functools.partial(_sc_scatter_body, NCH),
        out_type=(),
        mesh=plsc.VectorSubcoreMesh(core_axis_name="c",
                                    subcore_axis_name="s", num_cores=2,
                                    num_subcores=16),
        scratch_types=[
            pltpu.VMEM((NCH, CHW), jnp.int32),
            pltpu.VMEM((NCH, CHW), jnp.int32),
            pltpu.VMEM((NCH, CHW), jnp.float32),
            pltpu.SemaphoreType.DMA,
        ],
    )
    emb_ref = jax.new_ref(emb.reshape(-1))
    scatter(emb_ref, src3, dst3, vals.reshape(-1))
    return emb_ref[...].reshape(B, V)


# speedup vs baseline: 2.4853x; 1.5097x over previous
import functools
import math

import jax
import jax.numpy as jnp
from jax import lax
from jax.experimental import pallas as pl
from jax.experimental.pallas import tpu as pltpu
from jax.experimental.pallas import tpu_sc as plsc

SPW = 8
WIN = 256
NWK = 32
CHW = 128


def _attn_window_kernel(N, L, scale, starts_ref, rhs_hbm, lhs_ref, wq_ref,
                        wk_ref, wv_ref, wo_ref, pe_ref, lns_ref, lnb_ref,
                        wproj_ref, bproj_ref, woffid_ref, boffid_ref,
                        whead_ref, bhead_ref, vals_ref, proj_ref, xw_ref,
                        sem):
    w = pl.program_id(0)
    b0 = w * SPW
    s = jnp.minimum((starts_ref[b0] // 8) * 8, N - WIN)
    cp = pltpu.make_async_copy(rhs_hbm.at[pl.ds(s, WIN)], xw_ref, sem)
    cp.start()
    cp.wait()
    xw = xw_ref[...]
    row = lax.broadcasted_iota(jnp.int32, (WIN, 1), 0)
    roww = lax.broadcasted_iota(jnp.int32, (WIN, WIN), 0)
    colw = lax.broadcasted_iota(jnp.int32, (WIN, WIN), 1)
    mask = jnp.zeros((WIN, WIN), jnp.bool_)
    pos = jnp.zeros((WIN, 1), jnp.int32)
    lseg = jnp.zeros((WIN, 1), jnp.int32)
    for g in range(SPW):
        lo = starts_ref[b0 + g] - s
        hi = starts_ref[b0 + g + 1] - s
        rin = (row >= lo) & (row < hi)
        mask = mask | ((roww >= lo) & (roww < hi)
                       & (colw >= lo) & (colw < hi))
        pos = jnp.where(rin, row - lo, pos)
        lseg = jnp.where(rin, g, lseg)
    pos = jnp.minimum(pos, L - 1)
    lane = lax.broadcasted_iota(jnp.int32, (WIN, L), 1)
    oh_pos = (pos == lane).astype(jnp.float32)
    x = xw + jnp.dot(oh_pos, pe_ref[...],
                     preferred_element_type=jnp.float32)
    q = jnp.dot(x, wq_ref[...], preferred_element_type=jnp.float32)
    k = jnp.dot(x, wk_ref[...], preferred_element_type=jnp.float32)
    v = jnp.dot(x, wv_ref[...], preferred_element_type=jnp.float32)
    scores = lax.dot_general(q, k, (((1,), (1,)), ((), ())),
                             preferred_element_type=jnp.float32) * scale
    scores = jnp.where(mask, scores, -1e9)
    m = jnp.max(scores, axis=-1, keepdims=True)
    p = jnp.exp(scores - m)
    attn = p / jnp.sum(p, axis=-1, keepdims=True)
    out = jnp.dot(jnp.dot(attn, v, preferred_element_type=jnp.float32),
                  wo_ref[...], preferred_element_type=jnp.float32)
    h = xw + out
    mu = jnp.mean(h, axis=-1, keepdims=True)
    var = jnp.mean((h - mu) ** 2, axis=-1, keepdims=True)
    y = (h - mu) * lax.rsqrt(var + 1e-5) * lns_ref[...] + lnb_ref[...]
    lhs8 = lhs_ref[...]
    proj8 = jnp.dot(lhs8, wproj_ref[...],
                    preferred_element_type=jnp.float32) + bproj_ref[...]
    proj_ref[...] = proj8
    offid8 = jnp.dot(proj8, woffid_ref[...],
                     preferred_element_type=jnp.float32) + boffid_ref[0, 0]
    gl = lax.broadcasted_iota(jnp.int32, (WIN, SPW), 1)
    oh_seg = (lseg == gl).astype(jnp.float32)
    lhs_g = jnp.dot(oh_seg, lhs8, preferred_element_type=jnp.float32)
    offid_g = jnp.dot(oh_seg, offid8, preferred_element_type=jnp.float32)
    idgnn = (jnp.dot(y, whead_ref[...], preferred_element_type=jnp.float32)
             + bhead_ref[0, 0]
             + jnp.sum(lhs_g * y, axis=-1, keepdims=True)
             + offid_g)
    vals_ref[...] = idgnn


RING = 2
RB = 32


def _logits_kernel(NG, VLG, proj_ref, tab_ref, woffemb_ref, boffemb_ref,
                   out_hbm, bufs, sems):
    g = pl.program_id(0)
    slot = g % RING
    rows = RB * VLG

    @pl.when(g >= RING)
    def _():
        pltpu.make_async_copy(
            bufs.at[slot], out_hbm.at[pl.ds((g - RING) * rows, rows)],
            sems.at[slot]).wait()

    proj = proj_ref[...]
    logits = lax.dot_general(proj.astype(jnp.bfloat16), tab_ref[...],
                             (((1,), (1,)), ((), ())),
                             preferred_element_type=jnp.float32)
    off = jnp.dot(proj, woffemb_ref[...],
                  preferred_element_type=jnp.float32) + boffemb_ref[0, 0]
    bufs[slot] = (logits + off).reshape(rows, 128)
    pltpu.make_async_copy(bufs.at[slot], out_hbm.at[pl.ds(g * rows, rows)],
                          sems.at[slot]).start()

    @pl.when(g == NG - 1)
    def _():
        for s in range(RING):
            pltpu.make_async_copy(bufs.at[s], out_hbm.at[pl.ds(0, rows)],
                                  sems.at[s]).wait()


def _sc_scatter_body(nch, emb_ref, src_hbm, dst_hbm, vals_hbm,
                     src_v, dst_v, val_v, sem):
    c = lax.axis_index("c")
    s = lax.axis_index("s")
    wid = s * 2 + c
    pltpu.sync_copy(src_hbm.at[wid], src_v)
    pltpu.sync_copy(dst_hbm.at[wid], dst_v)
    for j in range(nch):
        pltpu.async_copy(vals_hbm.at[src_v.at[j]], val_v.at[j], sem).wait()
        pltpu.sync_copy(val_v.at[j], emb_ref.at[dst_v.at[j]])


def kernel(lhs_embedding, rhs_gnn_embedding, lhs_idgnn_batch,
           rhs_idgnn_index, rhs_table, W_q, W_k, W_v, W_o, pos_emb,
           ln_scale, ln_bias, W_proj, b_proj, W_off_emb, b_off_emb,
           W_off_id, b_off_id, W_head, b_head):
    B, C = lhs_embedding.shape
    N = rhs_gnn_embedding.shape[0]
    V, D = rhs_table.shape
    L = pos_emb.shape[0]
    NWIN = B // SPW

    seg = lhs_idgnn_batch.astype(jnp.int32)
    idx = rhs_idgnn_index.astype(jnp.int32)
    starts = jnp.searchsorted(
        seg, jnp.arange(B + 1, dtype=jnp.int32), side='left'
    ).astype(jnp.int32)

    full = lambda shp: pl.BlockSpec(shp, lambda w: tuple(0 for _ in shp))
    vals, proj = pl.pallas_call(
        functools.partial(_attn_window_kernel, N, L, 1.0 / math.sqrt(C)),
        grid=(NWIN,),
        in_specs=[
            pl.BlockSpec(memory_space=pltpu.SMEM),
            pl.BlockSpec(memory_space=pltpu.HBM),
            pl.BlockSpec((SPW, C), lambda w: (w, 0)),
            full((C, C)), full((C, C)), full((C, C)), full((C, C)),
            full((L, C)),
            full((1, C)), full((1, C)),
            full((C, D)), full((1, D)),
            full((D, 1)), full((1, 1)),
            full((C, 1)), full((1, 1)),
        ],
        out_specs=[
            pl.BlockSpec((WIN, 1), lambda w: (w, 0)),
            pl.BlockSpec((SPW, D), lambda w: (w, 0)),
        ],
        out_shape=[
            jax.ShapeDtypeStruct((NWIN * WIN, 1), jnp.float32),
            jax.ShapeDtypeStruct((B, D), jnp.float32),
        ],
        scratch_shapes=[pltpu.VMEM((WIN, C), jnp.float32),
                        pltpu.SemaphoreType.DMA],
        compiler_params=pltpu.CompilerParams(
            dimension_semantics=("arbitrary",)),
    )(starts, rhs_gnn_embedding, lhs_embedding, W_q, W_k, W_v, W_o,
      pos_emb, ln_scale.reshape(1, C), ln_bias.reshape(1, C), W_proj,
      b_proj.reshape(1, D), W_off_id.reshape(D, 1),
      b_off_id.reshape(1, 1), W_head.reshape(C, 1), b_head.reshape(1, 1))

    VLG = -(-V // 128)
    Vp = VLG * 128
    NG = B // RB
    tab_bf = jnp.pad(rhs_table, ((0, Vp - V), (0, 0))).astype(jnp.bfloat16)
    emb2 = pl.pallas_call(
        functools.partial(_logits_kernel, NG, VLG),
        grid=(NG,),
        in_specs=[
            pl.BlockSpec((RB, D), lambda g: (g, 0)),
            pl.BlockSpec((Vp, D), lambda g: (0, 0)),
            pl.BlockSpec((D, 1), lambda g: (0, 0)),
            pl.BlockSpec((1, 1), lambda g: (0, 0)),
        ],
        out_specs=pl.BlockSpec(memory_space=pl.ANY),
        out_shape=jax.ShapeDtypeStruct((B * VLG, 128), jnp.float32),
        scratch_shapes=[pltpu.VMEM((RING, RB * VLG, 128), jnp.float32),
                        pltpu.SemaphoreType.DMA((RING,))],
        compiler_params=pltpu.CompilerParams(
            dimension_semantics=("arbitrary",),
            vmem_limit_bytes=110 * 1024 * 1024),
    )(proj, tab_bf, W_off_emb.reshape(D, 1), b_off_emb.reshape(1, 1))

    n = jnp.arange(N, dtype=jnp.int32)
    w_of = seg // SPW
    ws = jnp.minimum((starts[w_of * SPW] // 8) * 8, N - WIN)
    src = w_of * WIN + n - ws
    dst = seg * Vp + idx
    NE = -(-N // (NWK * CHW)) * (NWK * CHW)
    if NE != N:
        pad = NE - N
        src = jnp.concatenate([src, jnp.broadcast_to(src[0], (pad,))])
        dst = jnp.concatenate([dst, jnp.broadcast_to(dst[0], (pad,))])
    NCH = NE // (NWK * CHW)
    src3 = src.reshape(NWK, NCH, CHW)
    dst3 = dst.reshape(NWK, NCH, CHW)

    scatter = pl.kernel(
        functools.partial(_sc_scatter_body, NCH),
        out_type=(),
        mesh=plsc.VectorSubcoreMesh(core_axis_name="c",
                                    subcore_axis_name="s", num_cores=2,
                                    num_subcores=16),
        scratch_types=[
            pltpu.VMEM((NCH, CHW), jnp.int32),
            pltpu.VMEM((NCH, CHW), jnp.int32),
            pltpu.VMEM((NCH, CHW), jnp.float32),
            pltpu.SemaphoreType.DMA,
        ],
    )
    emb_ref = jax.new_ref(emb2.reshape(-1))
    scatter(emb_ref, src3, dst3, vals.reshape(-1))
    return emb_ref[...].reshape(B, Vp)[:, :V]
